# baseline (speedup 1.0000x reference)
import functools

import jax
import jax.numpy as jnp
from jax.experimental import pallas as pl

B, IN, H, K, D = 16384, 128, 64, 1024, 8
BETA = 1e-3
G = H // D
BLK = 512


def _ln(t, g, b):
    m = jnp.mean(t, axis=-1, keepdims=True)
    v = jnp.mean((t - m) ** 2, axis=-1, keepdims=True)
    return (t - m) / jnp.sqrt(v + 1e-5) * g + b


def _body(x_ref, w1, b1, g1, be1, w2, b2, g2, be2, w3, b3, g3, be3, bd, cbt, cnorm_ref,
          dw1, db1, dg1, dbe1, dw2, db2, dg2, dbe2, dw3, db3,
          xhat_ref, loss_ref):
    f32 = jnp.float32
    x = x_ref[...]

    z = jax.nn.relu(_ln(jnp.dot(x, w1[...], preferred_element_type=f32) + b1[...], g1[...], be1[...]))
    z = jax.nn.relu(_ln(jnp.dot(z, w2[...], preferred_element_type=f32) + b2[...], g2[...], be2[...]))
    z = _ln(jnp.dot(z, w3[...], preferred_element_type=f32) + b3[...], g3[...], be3[...])

    dall = jnp.dot(z, bd[...], preferred_element_type=f32)
    cb_t = cbt[...]
    cn = cnorm_ref[...]
    lane = jax.lax.broadcasted_iota(jnp.int32, (x.shape[0], K), 1)
    zq_parts = []
    lsum = jnp.zeros((), dtype=f32)
    for gi in range(G):
        dg = dall[:, K * gi:K * (gi + 1)] + cn
        idx = jnp.argmin(dg, axis=1)
        onehot = jnp.where(lane == idx[:, None], 1.0, 0.0).astype(f32)
        zq_g = jax.lax.dot_general(onehot, cb_t, (((1,), (1,)), ((), ())),
                                   preferred_element_type=f32,
                                   precision=jax.lax.Precision.HIGHEST)
        diff = zq_g - z[:, D * gi:D * (gi + 1)]
        lsum = lsum + jnp.sum(diff * diff)
        zq_parts.append(zq_g)
    zq = jnp.concatenate(zq_parts, axis=1)

    h = jax.nn.relu(zq)
    h = jax.nn.relu(_ln(jnp.dot(h, dw1[...], preferred_element_type=f32) + db1[...], dg1[...], dbe1[...]))
    h = jax.nn.relu(_ln(jnp.dot(h, dw2[...], preferred_element_type=f32) + db2[...], dg2[...], dbe2[...]))
    xhat_ref[...] = jnp.dot(h, dw3[...], preferred_element_type=f32) + db3[...]

    lsum2d = lsum[None, None]

    @pl.when(pl.program_id(0) == 0)
    def _init():
        loss_ref[...] = lsum2d

    @pl.when(pl.program_id(0) != 0)
    def _acc():
        loss_ref[...] += lsum2d


@functools.partial(jax.jit, static_argnames=("interpret",))
def kernel(x, W1, b1, g1, be1, W2, b2, g2, be2, W3, b3, g3, be3, codebook,
           dW1, db1, dg1, dbe1, dW2, db2, dg2, dbe2, dW3, db3, interpret=False):
    row = lambda v: v.reshape(1, -1)
    cbt = codebook.T
    cnorm = jnp.sum(codebook * codebook, axis=1)[None, :]
    bd = jnp.kron(jnp.eye(G, dtype=jnp.float32), -2.0 * cbt)

    full = lambda a: pl.BlockSpec(a.shape, lambda i: (0,) * a.ndim)
    operands = [W1, row(b1), row(g1), row(be1), W2, row(b2), row(g2), row(be2),
                W3, row(b3), row(g3), row(be3), bd, cbt, cnorm,
                dW1, row(db1), row(dg1), row(dbe1), dW2, row(db2), row(dg2), row(dbe2),
                dW3, row(db3)]
    in_specs = [pl.BlockSpec((BLK, IN), lambda i: (i, 0))] + [full(a) for a in operands]

    xhat, lsum = pl.pallas_call(
        _body,
        grid=(B // BLK,),
        in_specs=in_specs,
        out_specs=[pl.BlockSpec((BLK, IN), lambda i: (i, 0)),
                   pl.BlockSpec((1, 1), lambda i: (0, 0))],
        out_shape=[jax.ShapeDtypeStruct((B, IN), jnp.float32),
                   jax.ShapeDtypeStruct((1, 1), jnp.float32)],
        interpret=interpret,
    )(x, *operands)

    loss = (lsum[0, 0] / (B * H)) * (1.0 + BETA)
    return (xhat, loss)


# speedup vs baseline: 1.0903x; 1.0903x over previous
import functools

import jax
import jax.numpy as jnp
from jax.experimental import pallas as pl

B, IN, H, K, D = 16384, 128, 64, 1024, 8
BETA = 1e-3
G = H // D
BLK = 512


def _ln(t, g, b):
    m = jnp.mean(t, axis=-1, keepdims=True)
    v = jnp.mean((t - m) ** 2, axis=-1, keepdims=True)
    return (t - m) / jnp.sqrt(v + 1e-5) * g + b


def _body(x_ref, w1, b1, g1, be1, w2, b2, g2, be2, w3, b3, g3, be3, bd, cb2_ref, cnorm_ref,
          dw1, db1, dg1, dbe1, dw2, db2, dg2, dbe2, dw3, db3,
          xhat_ref, loss_ref):
    f32 = jnp.float32
    x = x_ref[...]

    z = jax.nn.relu(_ln(jnp.dot(x, w1[...], preferred_element_type=f32) + b1[...], g1[...], be1[...]))
    z = jax.nn.relu(_ln(jnp.dot(z, w2[...], preferred_element_type=f32) + b2[...], g2[...], be2[...]))
    z = _ln(jnp.dot(z, w3[...], preferred_element_type=f32) + b3[...], g3[...], be3[...])

    dall = jnp.dot(z, bd[...], preferred_element_type=f32)
    cn = cnorm_ref[...]
    lane = jax.lax.broadcasted_iota(jnp.int32, (x.shape[0], K), 1)
    zq_parts = []
    lsum = jnp.zeros((), dtype=f32)
    for gi in range(G):
        dg = dall[:, K * gi:K * (gi + 1)] + cn
        idx = jnp.argmin(dg, axis=1)
        onehot = jnp.where(lane == idx[:, None], 1.0, 0.0).astype(jnp.bfloat16)
        two = jax.lax.dot_general(onehot, cb2_ref[...], (((1,), (1,)), ((), ())),
                                  preferred_element_type=f32)
        zq_g = two[:, :D] + two[:, D:]
        diff = zq_g - z[:, D * gi:D * (gi + 1)]
        lsum = lsum + jnp.sum(diff * diff)
        zq_parts.append(zq_g)
    zq = jnp.concatenate(zq_parts, axis=1)

    h = jax.nn.relu(zq)
    h = jax.nn.relu(_ln(jnp.dot(h, dw1[...], preferred_element_type=f32) + db1[...], dg1[...], dbe1[...]))
    h = jax.nn.relu(_ln(jnp.dot(h, dw2[...], preferred_element_type=f32) + db2[...], dg2[...], dbe2[...]))
    xhat_ref[...] = jnp.dot(h, dw3[...], preferred_element_type=f32) + db3[...]

    lsum2d = lsum[None, None]

    @pl.when(pl.program_id(0) == 0)
    def _init():
        loss_ref[...] = lsum2d

    @pl.when(pl.program_id(0) != 0)
    def _acc():
        loss_ref[...] += lsum2d


@functools.partial(jax.jit, static_argnames=("interpret",))
def kernel(x, W1, b1, g1, be1, W2, b2, g2, be2, W3, b3, g3, be3, codebook,
           dW1, db1, dg1, dbe1, dW2, db2, dg2, dbe2, dW3, db3, interpret=False):
    row = lambda v: v.reshape(1, -1)
    cbt = codebook.T
    cnorm = jnp.sum(codebook * codebook, axis=1)[None, :]
    bd = jnp.kron(jnp.eye(G, dtype=jnp.float32), -2.0 * cbt)
    cbt_hi = cbt.astype(jnp.bfloat16)
    cbt_lo = (cbt - cbt_hi.astype(jnp.float32)).astype(jnp.bfloat16)
    cb2 = jnp.concatenate([cbt_hi, cbt_lo], axis=0)

    full = lambda a: pl.BlockSpec(a.shape, lambda i: (0,) * a.ndim)
    operands = [W1, row(b1), row(g1), row(be1), W2, row(b2), row(g2), row(be2),
                W3, row(b3), row(g3), row(be3), bd, cb2, cnorm,
                dW1, row(db1), row(dg1), row(dbe1), dW2, row(db2), row(dg2), row(dbe2),
                dW3, row(db3)]
    in_specs = [pl.BlockSpec((BLK, IN), lambda i: (i, 0))] + [full(a) for a in operands]

    xhat, lsum = pl.pallas_call(
        _body,
        grid=(B // BLK,),
        in_specs=in_specs,
        out_specs=[pl.BlockSpec((BLK, IN), lambda i: (i, 0)),
                   pl.BlockSpec((1, 1), lambda i: (0, 0))],
        out_shape=[jax.ShapeDtypeStruct((B, IN), jnp.float32),
                   jax.ShapeDtypeStruct((1, 1), jnp.float32)],
        interpret=interpret,
    )(x, *operands)

    loss = (lsum[0, 0] / (B * H)) * (1.0 + BETA)
    return (xhat, loss)


# speedup vs baseline: 1.5274x; 1.4009x over previous
import functools

import jax
import jax.numpy as jnp
from jax.experimental import pallas as pl

B, IN, H, K, D = 16384, 128, 64, 1024, 8
BETA = 1e-3
G = H // D
BLK = 512


def _ln(t, g, b):
    m = jnp.mean(t, axis=-1, keepdims=True)
    v = jnp.mean((t - m) ** 2, axis=-1, keepdims=True)
    return (t - m) / jnp.sqrt(v + 1e-5) * g + b


def _body(x_ref, w1, b1, g1, be1, w2, b2, g2, be2, w3, b3, g3, be3, bd, cb2_ref, cnorm_ref,
          dw1, db1, dg1, dbe1, dw2, db2, dg2, dbe2, dw3, db3,
          xhat_ref, loss_ref):
    f32 = jnp.float32
    x = x_ref[...]

    z = jax.nn.relu(_ln(jnp.dot(x, w1[...], preferred_element_type=f32) + b1[...], g1[...], be1[...]))
    z = jax.nn.relu(_ln(jnp.dot(z, w2[...], preferred_element_type=f32) + b2[...], g2[...], be2[...]))
    z = _ln(jnp.dot(z, w3[...], preferred_element_type=f32) + b3[...], g3[...], be3[...])

    dall = jnp.dot(z, bd[...], preferred_element_type=f32)
    cn = cnorm_ref[...]
    lane = jax.lax.broadcasted_iota(jnp.int32, (x.shape[0], K), 1)
    zq_parts = []
    lparts = []
    for gi in range(G):
        dg = dall[:, K * gi:K * (gi + 1)] + cn
        idx = jnp.argmin(dg, axis=1)
        onehot = jnp.where(lane == idx[:, None], 1.0, 0.0).astype(jnp.bfloat16)
        two = jax.lax.dot_general(onehot, cb2_ref[...], (((1,), (1,)), ((), ())),
                                  preferred_element_type=f32)
        zq_g = two[:, :D] + two[:, D:]
        diff = zq_g - z[:, D * gi:D * (gi + 1)]
        lparts.append(jnp.sum(diff * diff, axis=1, keepdims=True))
        zq_parts.append(zq_g)
    zq = jnp.concatenate(zq_parts, axis=1)
    lsum = jnp.sum(jnp.concatenate(lparts, axis=1))

    h = jax.nn.relu(zq)
    h = jax.nn.relu(_ln(jnp.dot(h, dw1[...], preferred_element_type=f32) + db1[...], dg1[...], dbe1[...]))
    h = jax.nn.relu(_ln(jnp.dot(h, dw2[...], preferred_element_type=f32) + db2[...], dg2[...], dbe2[...]))
    xhat_ref[...] = jnp.dot(h, dw3[...], preferred_element_type=f32) + db3[...]

    lsum2d = lsum[None, None]

    @pl.when(pl.program_id(0) == 0)
    def _init():
        loss_ref[...] = lsum2d

    @pl.when(pl.program_id(0) != 0)
    def _acc():
        loss_ref[...] += lsum2d


@functools.partial(jax.jit, static_argnames=("interpret",))
def kernel(x, W1, b1, g1, be1, W2, b2, g2, be2, W3, b3, g3, be3, codebook,
           dW1, db1, dg1, dbe1, dW2, db2, dg2, dbe2, dW3, db3, interpret=False):
    row = lambda v: v.reshape(1, -1)
    cbt = codebook.T
    cnorm = jnp.sum(codebook * codebook, axis=1)[None, :]
    bd = jnp.kron(jnp.eye(G, dtype=jnp.float32), -2.0 * cbt)
    cbt_hi = cbt.astype(jnp.bfloat16)
    cbt_lo = (cbt - cbt_hi.astype(jnp.float32)).astype(jnp.bfloat16)
    cb2 = jnp.concatenate([cbt_hi, cbt_lo], axis=0)

    full = lambda a: pl.BlockSpec(a.shape, lambda i: (0,) * a.ndim)
    operands = [W1, row(b1), row(g1), row(be1), W2, row(b2), row(g2), row(be2),
                W3, row(b3), row(g3), row(be3), bd, cb2, cnorm,
                dW1, row(db1), row(dg1), row(dbe1), dW2, row(db2), row(dg2), row(dbe2),
                dW3, row(db3)]
    in_specs = [pl.BlockSpec((BLK, IN), lambda i: (i, 0))] + [full(a) for a in operands]

    xhat, lsum = pl.pallas_call(
        _body,
        grid=(B // BLK,),
        in_specs=in_specs,
        out_specs=[pl.BlockSpec((BLK, IN), lambda i: (i, 0)),
                   pl.BlockSpec((1, 1), lambda i: (0, 0))],
        out_shape=[jax.ShapeDtypeStruct((B, IN), jnp.float32),
                   jax.ShapeDtypeStruct((1, 1), jnp.float32)],
        interpret=interpret,
    )(x, *operands)

    loss = (lsum[0, 0] / (B * H)) * (1.0 + BETA)
    return (xhat, loss)


# speedup vs baseline: 2.3721x; 1.5530x over previous
import functools

import jax
import jax.numpy as jnp
from jax.experimental import pallas as pl

B, IN, H, K, D = 16384, 128, 64, 1024, 8
BETA = 1e-3
G = H // D
BLK = 512


def _ln(t, g, b):
    m = jnp.mean(t, axis=-1, keepdims=True)
    v = jnp.mean((t - m) ** 2, axis=-1, keepdims=True)
    return (t - m) / jnp.sqrt(v + 1e-5) * g + b


def _body(x_ref, w1, b1, g1, be1, w2, b2, g2, be2, w3, b3, g3, be3, bd, cb2_ref, cnorm_ref,
          dw1, db1, dg1, dbe1, dw2, db2, dg2, dbe2, dw3, db3,
          xhat_ref, loss_ref):
    f32 = jnp.float32
    x = x_ref[...]

    z = jax.nn.relu(_ln(jnp.dot(x, w1[...], preferred_element_type=f32) + b1[...], g1[...], be1[...]))
    z = jax.nn.relu(_ln(jnp.dot(z, w2[...], preferred_element_type=f32) + b2[...], g2[...], be2[...]))
    z = _ln(jnp.dot(z, w3[...], preferred_element_type=f32) + b3[...], g3[...], be3[...])

    dall = jnp.dot(z, bd[...], preferred_element_type=f32)
    cn = cnorm_ref[...]
    lane = jax.lax.broadcasted_iota(jnp.int32, (x.shape[0], K), 1)
    oh_parts = []
    for gi in range(G):
        dg = dall[:, K * gi:K * (gi + 1)] + cn
        idx = jnp.argmin(dg, axis=1)
        oh_parts.append(
            jnp.where(lane == idx[:, None], 1.0, 0.0).astype(jnp.bfloat16))
    onehot = jnp.concatenate(oh_parts, axis=1)
    two = jnp.dot(onehot, cb2_ref[...], preferred_element_type=f32)
    zq = two[:, :H] + two[:, H:]
    diff = zq - z
    lsum = jnp.sum(diff * diff)

    h = jax.nn.relu(zq)
    h = jax.nn.relu(_ln(jnp.dot(h, dw1[...], preferred_element_type=f32) + db1[...], dg1[...], dbe1[...]))
    h = jax.nn.relu(_ln(jnp.dot(h, dw2[...], preferred_element_type=f32) + db2[...], dg2[...], dbe2[...]))
    xhat_ref[...] = jnp.dot(h, dw3[...], preferred_element_type=f32) + db3[...]

    lsum2d = lsum[None, None]

    @pl.when(pl.program_id(0) == 0)
    def _init():
        loss_ref[...] = lsum2d

    @pl.when(pl.program_id(0) != 0)
    def _acc():
        loss_ref[...] += lsum2d


@functools.partial(jax.jit, static_argnames=("interpret",))
def kernel(x, W1, b1, g1, be1, W2, b2, g2, be2, W3, b3, g3, be3, codebook,
           dW1, db1, dg1, dbe1, dW2, db2, dg2, dbe2, dW3, db3, interpret=False):
    row = lambda v: v.reshape(1, -1)
    cbt = codebook.T
    cnorm = jnp.sum(codebook * codebook, axis=1)[None, :]
    bd = jnp.kron(jnp.eye(G, dtype=jnp.float32), -2.0 * cbt)
    cb_hi = codebook.astype(jnp.bfloat16).astype(jnp.float32)
    cb_lo = codebook - cb_hi
    eye = jnp.eye(G, dtype=jnp.float32)
    cb2 = jnp.concatenate([jnp.kron(eye, cb_hi), jnp.kron(eye, cb_lo)],
                          axis=1).astype(jnp.bfloat16)

    full = lambda a: pl.BlockSpec(a.shape, lambda i: (0,) * a.ndim)
    operands = [W1, row(b1), row(g1), row(be1), W2, row(b2), row(g2), row(be2),
                W3, row(b3), row(g3), row(be3), bd, cb2, cnorm,
                dW1, row(db1), row(dg1), row(dbe1), dW2, row(db2), row(dg2), row(dbe2),
                dW3, row(db3)]
    in_specs = [pl.BlockSpec((BLK, IN), lambda i: (i, 0))] + [full(a) for a in operands]

    xhat, lsum = pl.pallas_call(
        _body,
        grid=(B // BLK,),
        in_specs=in_specs,
        out_specs=[pl.BlockSpec((BLK, IN), lambda i: (i, 0)),
                   pl.BlockSpec((1, 1), lambda i: (0, 0))],
        out_shape=[jax.ShapeDtypeStruct((B, IN), jnp.float32),
                   jax.ShapeDtypeStruct((1, 1), jnp.float32)],
        interpret=interpret,
    )(x, *operands)

    loss = (lsum[0, 0] / (B * H)) * (1.0 + BETA)
    return (xhat, loss)


# speedup vs baseline: 2.5101x; 1.0582x over previous
import functools

import jax
import jax.numpy as jnp
from jax.experimental import pallas as pl

B, IN, H, K, D = 16384, 128, 64, 1024, 8
BETA = 1e-3
G = H // D
BLK = 1024


def _ln(t, g, b):
    m = jnp.mean(t, axis=-1, keepdims=True)
    v = jnp.mean((t - m) ** 2, axis=-1, keepdims=True)
    return (t - m) / jnp.sqrt(v + 1e-5) * g + b


def _body(x_ref, w1, b1, g1, be1, w2, b2, g2, be2, w3, b3, g3, be3, bd, cb2_ref, cnorm_ref,
          dw1, db1, dg1, dbe1, dw2, db2, dg2, dbe2, dw3, db3,
          xhat_ref, loss_ref):
    f32 = jnp.float32
    x = x_ref[...]

    z = jax.nn.relu(_ln(jnp.dot(x, w1[...], preferred_element_type=f32) + b1[...], g1[...], be1[...]))
    z = jax.nn.relu(_ln(jnp.dot(z, w2[...], preferred_element_type=f32) + b2[...], g2[...], be2[...]))
    z = _ln(jnp.dot(z, w3[...], preferred_element_type=f32) + b3[...], g3[...], be3[...])

    dall = jnp.dot(z, bd[...], preferred_element_type=f32)
    cn = cnorm_ref[...]
    lane = jax.lax.broadcasted_iota(jnp.int32, (x.shape[0], K), 1)
    oh_parts = []
    for gi in range(G):
        dg = dall[:, K * gi:K * (gi + 1)] + cn
        idx = jnp.argmin(dg, axis=1)
        oh_parts.append(
            jnp.where(lane == idx[:, None], 1.0, 0.0).astype(jnp.bfloat16))
    onehot = jnp.concatenate(oh_parts, axis=1)
    two = jnp.dot(onehot, cb2_ref[...], preferred_element_type=f32)
    zq = two[:, :H] + two[:, H:]
    diff = zq - z
    lsum = jnp.sum(diff * diff)

    h = jax.nn.relu(zq)
    h = jax.nn.relu(_ln(jnp.dot(h, dw1[...], preferred_element_type=f32) + db1[...], dg1[...], dbe1[...]))
    h = jax.nn.relu(_ln(jnp.dot(h, dw2[...], preferred_element_type=f32) + db2[...], dg2[...], dbe2[...]))
    xhat_ref[...] = jnp.dot(h, dw3[...], preferred_element_type=f32) + db3[...]

    lsum2d = lsum[None, None]

    @pl.when(pl.program_id(0) == 0)
    def _init():
        loss_ref[...] = lsum2d

    @pl.when(pl.program_id(0) != 0)
    def _acc():
        loss_ref[...] += lsum2d


@functools.partial(jax.jit, static_argnames=("interpret",))
def kernel(x, W1, b1, g1, be1, W2, b2, g2, be2, W3, b3, g3, be3, codebook,
           dW1, db1, dg1, dbe1, dW2, db2, dg2, dbe2, dW3, db3, interpret=False):
    row = lambda v: v.reshape(1, -1)
    cbt = codebook.T
    cnorm = jnp.sum(codebook * codebook, axis=1)[None, :]
    bd = jnp.kron(jnp.eye(G, dtype=jnp.float32), -2.0 * cbt)
    cb_hi = codebook.astype(jnp.bfloat16).astype(jnp.float32)
    cb_lo = codebook - cb_hi
    eye = jnp.eye(G, dtype=jnp.float32)
    cb2 = jnp.concatenate([jnp.kron(eye, cb_hi), jnp.kron(eye, cb_lo)],
                          axis=1).astype(jnp.bfloat16)

    full = lambda a: pl.BlockSpec(a.shape, lambda i: (0,) * a.ndim)
    operands = [W1, row(b1), row(g1), row(be1), W2, row(b2), row(g2), row(be2),
                W3, row(b3), row(g3), row(be3), bd, cb2, cnorm,
                dW1, row(db1), row(dg1), row(dbe1), dW2, row(db2), row(dg2), row(dbe2),
                dW3, row(db3)]
    in_specs = [pl.BlockSpec((BLK, IN), lambda i: (i, 0))] + [full(a) for a in operands]

    xhat, lsum = pl.pallas_call(
        _body,
        grid=(B // BLK,),
        in_specs=in_specs,
        out_specs=[pl.BlockSpec((BLK, IN), lambda i: (i, 0)),
                   pl.BlockSpec((1, 1), lambda i: (0, 0))],
        out_shape=[jax.ShapeDtypeStruct((B, IN), jnp.float32),
                   jax.ShapeDtypeStruct((1, 1), jnp.float32)],
        interpret=interpret,
    )(x, *operands)

    loss = (lsum[0, 0] / (B * H)) * (1.0 + BETA)
    return (xhat, loss)


# speedup vs baseline: 2.6154x; 1.0420x over previous
import functools

import jax
import jax.numpy as jnp
from jax.experimental import pallas as pl

B, IN, H, K, D = 16384, 128, 64, 1024, 8
BETA = 1e-3
G = H // D
BLK = 1024


def _ln(t, g, b):
    m = jnp.mean(t, axis=-1, keepdims=True)
    v = jnp.mean((t - m) ** 2, axis=-1, keepdims=True)
    return (t - m) / jnp.sqrt(v + 1e-5) * g + b


def _body(x_ref, w1, b1, g1, be1, w2, b2, g2, be2, w3, b3, g3, be3, bd, cb2_ref, cnorm_ref,
          dw1, db1, dg1, dbe1, dw2, db2, dg2, dbe2, dw3, db3,
          xhat_ref, loss_ref):
    f32 = jnp.float32
    x = x_ref[...]

    z = jax.nn.relu(_ln(jnp.dot(x, w1[...], preferred_element_type=f32) + b1[...], g1[...], be1[...]))
    z = jax.nn.relu(_ln(jnp.dot(z, w2[...], preferred_element_type=f32) + b2[...], g2[...], be2[...]))
    z = _ln(jnp.dot(z, w3[...], preferred_element_type=f32) + b3[...], g3[...], be3[...])

    dall = jnp.dot(z, bd[...], preferred_element_type=f32)
    cn = cnorm_ref[...]
    oh_parts = []
    for gi in range(G):
        dg = dall[:, K * gi:K * (gi + 1)] + cn
        dmin = jnp.min(dg, axis=1, keepdims=True)
        oh_parts.append(
            jnp.where(dg == dmin, 1.0, 0.0).astype(jnp.bfloat16))
    onehot = jnp.concatenate(oh_parts, axis=1)
    two = jnp.dot(onehot, cb2_ref[...], preferred_element_type=f32)
    zq = two[:, :H] + two[:, H:]
    diff = zq - z
    lsum = jnp.sum(diff * diff)

    h = jax.nn.relu(zq)
    h = jax.nn.relu(_ln(jnp.dot(h, dw1[...], preferred_element_type=f32) + db1[...], dg1[...], dbe1[...]))
    h = jax.nn.relu(_ln(jnp.dot(h, dw2[...], preferred_element_type=f32) + db2[...], dg2[...], dbe2[...]))
    xhat_ref[...] = jnp.dot(h, dw3[...], preferred_element_type=f32) + db3[...]

    lsum2d = lsum[None, None]

    @pl.when(pl.program_id(0) == 0)
    def _init():
        loss_ref[...] = lsum2d

    @pl.when(pl.program_id(0) != 0)
    def _acc():
        loss_ref[...] += lsum2d


@functools.partial(jax.jit, static_argnames=("interpret",))
def kernel(x, W1, b1, g1, be1, W2, b2, g2, be2, W3, b3, g3, be3, codebook,
           dW1, db1, dg1, dbe1, dW2, db2, dg2, dbe2, dW3, db3, interpret=False):
    row = lambda v: v.reshape(1, -1)
    cbt = codebook.T
    cnorm = jnp.sum(codebook * codebook, axis=1)[None, :]
    bd = jnp.kron(jnp.eye(G, dtype=jnp.float32), -2.0 * cbt)
    cb_hi = codebook.astype(jnp.bfloat16).astype(jnp.float32)
    cb_lo = codebook - cb_hi
    eye = jnp.eye(G, dtype=jnp.float32)
    cb2 = jnp.concatenate([jnp.kron(eye, cb_hi), jnp.kron(eye, cb_lo)],
                          axis=1).astype(jnp.bfloat16)

    full = lambda a: pl.BlockSpec(a.shape, lambda i: (0,) * a.ndim)
    operands = [W1, row(b1), row(g1), row(be1), W2, row(b2), row(g2), row(be2),
                W3, row(b3), row(g3), row(be3), bd, cb2, cnorm,
                dW1, row(db1), row(dg1), row(dbe1), dW2, row(db2), row(dg2), row(dbe2),
                dW3, row(db3)]
    in_specs = [pl.BlockSpec((BLK, IN), lambda i: (i, 0))] + [full(a) for a in operands]

    xhat, lsum = pl.pallas_call(
        _body,
        grid=(B // BLK,),
        in_specs=in_specs,
        out_specs=[pl.BlockSpec((BLK, IN), lambda i: (i, 0)),
                   pl.BlockSpec((1, 1), lambda i: (0, 0))],
        out_shape=[jax.ShapeDtypeStruct((B, IN), jnp.float32),
                   jax.ShapeDtypeStruct((1, 1), jnp.float32)],
        interpret=interpret,
    )(x, *operands)

    loss = (lsum[0, 0] / (B * H)) * (1.0 + BETA)
    return (xhat, loss)


# speedup vs baseline: 2.6224x; 1.0027x over previous
import jax
import jax.numpy as jnp
from jax.experimental import pallas as pl

B, IN, H, K, D = 16384, 128, 64, 1024, 8
BETA = 1e-3
G = H // D
BLK = 1024


def _ln(t, g, b):
    m = jnp.mean(t, axis=-1, keepdims=True)
    v = jnp.mean((t - m) ** 2, axis=-1, keepdims=True)
    return (t - m) / jnp.sqrt(v + 1e-5) * g + b


def _body(x_ref, w1, b1, g1, be1, w2, b2, g2, be2, w3, b3, g3, be3, bd, cb2_ref, cnorm_ref,
          dw1, db1, dg1, dbe1, dw2, db2, dg2, dbe2, dw3, db3,
          xhat_ref, loss_ref):
    f32 = jnp.float32
    x = x_ref[...]

    z = jax.nn.relu(_ln(jnp.dot(x, w1[...], preferred_element_type=f32) + b1[...], g1[...], be1[...]))
    z = jax.nn.relu(_ln(jnp.dot(z, w2[...], preferred_element_type=f32) + b2[...], g2[...], be2[...]))
    z = _ln(jnp.dot(z, w3[...], preferred_element_type=f32) + b3[...], g3[...], be3[...])

    dall = jnp.dot(z, bd[...], preferred_element_type=f32)
    cn = cnorm_ref[...]
    oh_parts = []
    for gi in range(G):
        dg = dall[:, K * gi:K * (gi + 1)] + cn
        dmin = jnp.min(dg, axis=1, keepdims=True)
        oh_parts.append(
            jnp.where(dg == dmin, 1.0, 0.0).astype(jnp.bfloat16))
    onehot = jnp.concatenate(oh_parts, axis=1)
    two = jnp.dot(onehot, cb2_ref[...], preferred_element_type=f32)
    zq = two[:, :H] + two[:, H:]
    diff = zq - z
    lsum = jnp.sum(diff * diff)

    h = jax.nn.relu(zq)
    h = jax.nn.relu(_ln(jnp.dot(h, dw1[...], preferred_element_type=f32) + db1[...], dg1[...], dbe1[...]))
    h = jax.nn.relu(_ln(jnp.dot(h, dw2[...], preferred_element_type=f32) + db2[...], dg2[...], dbe2[...]))
    xhat_ref[...] = jnp.dot(h, dw3[...], preferred_element_type=f32) + db3[...]

    lsum2d = lsum[None, None]

    @pl.when(pl.program_id(0) == 0)
    def _init():
        loss_ref[...] = lsum2d

    @pl.when(pl.program_id(0) != 0)
    def _acc():
        loss_ref[...] += lsum2d


@jax.jit
def kernel(x, W1, b1, g1, be1, W2, b2, g2, be2, W3, b3, g3, be3, codebook,
           dW1, db1, dg1, dbe1, dW2, db2, dg2, dbe2, dW3, db3):
    row = lambda v: v.reshape(1, -1)
    cbt = codebook.T
    cnorm = jnp.sum(codebook * codebook, axis=1)[None, :]
    bd = jnp.kron(jnp.eye(G, dtype=jnp.float32), -2.0 * cbt)
    cb_hi = codebook.astype(jnp.bfloat16).astype(jnp.float32)
    cb_lo = codebook - cb_hi
    eye = jnp.eye(G, dtype=jnp.float32)
    cb2 = jnp.concatenate([jnp.kron(eye, cb_hi), jnp.kron(eye, cb_lo)],
                          axis=1).astype(jnp.bfloat16)

    full = lambda a: pl.BlockSpec(a.shape, lambda i: (0,) * a.ndim)
    operands = [W1, row(b1), row(g1), row(be1), W2, row(b2), row(g2), row(be2),
                W3, row(b3), row(g3), row(be3), bd, cb2, cnorm,
                dW1, row(db1), row(dg1), row(dbe1), dW2, row(db2), row(dg2), row(dbe2),
                dW3, row(db3)]
    in_specs = [pl.BlockSpec((BLK, IN), lambda i: (i, 0))] + [full(a) for a in operands]

    xhat, lsum = pl.pallas_call(
        _body,
        grid=(B // BLK,),
        in_specs=in_specs,
        out_specs=[pl.BlockSpec((BLK, IN), lambda i: (i, 0)),
                   pl.BlockSpec((1, 1), lambda i: (0, 0))],
        out_shape=[jax.ShapeDtypeStruct((B, IN), jnp.float32),
                   jax.ShapeDtypeStruct((1, 1), jnp.float32)],
    )(x, *operands)

    loss = (lsum[0, 0] / (B * H)) * (1.0 + BETA)
    return (xhat, loss)
